# Initial kernel scaffold; baseline (speedup 1.0000x reference)
import jax
import jax.numpy as jnp
from jax.experimental import pallas as pl


def kernel(X, adj, W_heads, a_heads, W_out, a_out):
    raise NotImplementedError("write your pallas kernel here")


# speedup vs baseline: 5.4941x; 5.4941x over previous
import functools

import jax
import jax.numpy as jnp
from jax import lax
from jax.experimental import pallas as pl
from jax.experimental.pallas import tpu as pltpu
from jax.experimental.pallas import tpu_sc as plsc

ALPHA = 0.3
NH = 8
HID = 16
NOUT = 100
N = 10000
NF = 128
E = 320000

NC = 2
NS = 16
AW = 80
EPT = E // NS
B = 80
RCH = 80
NRCH = N // RCH


def _elu(x):
    return jnp.where(x > 0, x, jnp.exp(x) - 1.0)


def _prep1_body(x_ref, w_ref, a1_ref, a2_ref, haug_ref, s1p_ref):
    h = jnp.dot(x_ref[...], w_ref[...], preferred_element_type=jnp.float32)
    s1 = jnp.dot(h, a1_ref[...], preferred_element_type=jnp.float32)
    s2 = jnp.dot(h, a2_ref[...], preferred_element_type=jnp.float32)
    z12 = jnp.zeros((h.shape[0], 12), jnp.float32)
    haug_ref[0] = jnp.concatenate([h[:, :64], s2[:, :4], z12], axis=1)
    haug_ref[1] = jnp.concatenate([h[:, 64:], s2[:, 4:], z12], axis=1)
    s1p_ref[0] = jnp.concatenate([s1[:, :4], z12], axis=1)
    s1p_ref[1] = jnp.concatenate([s1[:, 4:], z12], axis=1)


def _prep1(X, W_all, A1, A2):
    bl = 1000
    return pl.pallas_call(
        _prep1_body,
        grid=(N // bl,),
        in_specs=[
            pl.BlockSpec((bl, NF), lambda i: (i, 0)),
            pl.BlockSpec((NF, NF), lambda i: (0, 0)),
            pl.BlockSpec((NF, NH), lambda i: (0, 0)),
            pl.BlockSpec((NF, NH), lambda i: (0, 0)),
        ],
        out_specs=[
            pl.BlockSpec((NC, bl, AW), lambda i: (0, i, 0)),
            pl.BlockSpec((NC, bl, 16), lambda i: (0, i, 0)),
        ],
        out_shape=[
            jax.ShapeDtypeStruct((NC, N, AW), jnp.float32),
            jax.ShapeDtypeStruct((NC, N, 16), jnp.float32),
        ],
    )(X, W_all, A1, A2)


def _mid_body(acc_ref, wout_ref, a1_ref, a2_ref, haug_ref, s1p_ref):
    acc = acc_ref[...]
    parts = []
    for c in range(2):
        for hh in range(4):
            parts.append(acc[c, :, 16 * hh:16 * hh + 16]
                         / acc[c, :, 64 + hh:65 + hh])
    xc = _elu(jnp.concatenate(parts, axis=1))
    h2 = jnp.dot(xc, wout_ref[...], preferred_element_type=jnp.float32)
    s1b = jnp.dot(h2, a1_ref[...], preferred_element_type=jnp.float32)
    s2b = jnp.dot(h2, a2_ref[...], preferred_element_type=jnp.float32)
    nrow = h2.shape[0]
    s1b4 = jnp.broadcast_to(s1b, (nrow, 4))
    s2b4 = jnp.broadcast_to(s2b, (nrow, 4))
    z12 = jnp.zeros((nrow, 12), jnp.float32)
    z28 = jnp.zeros((nrow, 28), jnp.float32)
    haug_ref[0] = jnp.concatenate([h2[:, :64], s2b4, z12], axis=1)
    haug_ref[1] = jnp.concatenate([h2[:, 64:100], z28, s2b4, z12], axis=1)
    s1p_ref[0] = jnp.concatenate([s1b4, z12], axis=1)
    s1p_ref[1] = jnp.concatenate([s1b4, z12], axis=1)


def _mid(acc1, W_out, a1c, a2c):
    bl = 1000
    return pl.pallas_call(
        _mid_body,
        grid=(N // bl,),
        in_specs=[
            pl.BlockSpec((NC, bl, AW), lambda i: (0, i, 0)),
            pl.BlockSpec((NF, NOUT), lambda i: (0, 0)),
            pl.BlockSpec((NOUT, 1), lambda i: (0, 0)),
            pl.BlockSpec((NOUT, 1), lambda i: (0, 0)),
        ],
        out_specs=[
            pl.BlockSpec((NC, bl, AW), lambda i: (0, i, 0)),
            pl.BlockSpec((NC, bl, 16), lambda i: (0, i, 0)),
        ],
        out_shape=[
            jax.ShapeDtypeStruct((NC, N, AW), jnp.float32),
            jax.ShapeDtypeStruct((NC, N, 16), jnp.float32),
        ],
    )(acc1, W_out, a1c, a2c)


def _fin_body(acc_ref, out_ref):
    acc = acc_ref[...]
    o0 = acc[0, :, :64] / acc[0, :, 64:65]
    o1 = acc[1, :, :36] / acc[1, :, 64:65]
    out_ref[...] = _elu(jnp.concatenate([o0, o1], axis=1))


def _fin(acc2):
    bl = 1000
    return pl.pallas_call(
        _fin_body,
        grid=(N // bl,),
        in_specs=[pl.BlockSpec((NC, bl, AW), lambda i: (0, i, 0))],
        out_specs=pl.BlockSpec((bl, NOUT), lambda i: (i, 0)),
        out_shape=jax.ShapeDtypeStruct((N, NOUT), jnp.float32),
    )(acc2)


def _sc_edges_body(haug_hbm, s1p_hbm, src_hbm, dst_hbm, acc_out_hbm,
                   acc_sh, s1buf, sidx, didx, sidx_adj, didx_adj, grow, srow,
                   zbuf):
    c = lax.axis_index("c")
    s = lax.axis_index("s")
    coff = c * N

    zf = jnp.zeros((16,), jnp.float32)
    for r in range(RCH):
        for q in range(AW // 16):
            zbuf[r, pl.ds(q * 16, 16)] = zf
    nch = jnp.where(s <= (NRCH % NS) - 1, NRCH // NS + 1, NRCH // NS)

    def zero_chunk(k, carry):
        ch = s + k * NS
        pltpu.sync_copy(zbuf, acc_sh.at[pl.ds(ch * RCH, RCH)])
        return carry

    lax.fori_loop(0, nch, zero_chunk, 0)
    plsc.subcore_barrier()

    iota16 = lax.iota(jnp.int32, 16)

    def chunk(i, carry):
        base = s * EPT + i * B
        pltpu.sync_copy(src_hbm.at[pl.ds(base, B)], sidx)
        pltpu.sync_copy(dst_hbm.at[pl.ds(base, B)], didx)
        for j in range(B // 16):
            sl = pl.ds(j * 16, 16)
            didx_adj[sl] = didx[sl] + coff
            sidx_adj[sl] = sidx[sl] + coff
        pltpu.sync_copy(haug_hbm.at[didx_adj], grow)
        pltpu.sync_copy(s1p_hbm.at[sidx_adj], s1buf)
        for j in range(B // 16):
            rowv = j * 16 + iota16
            for hh in range(4):
                hsel = jnp.full((16,), hh, jnp.int32)
                wcol = jnp.full((16,), 64 + hh, jnp.int32)
                s1v = plsc.load_gather(s1buf, [rowv, hsel])
                s2v = plsc.load_gather(grow, [rowv, wcol])
                x = s1v + s2v
                w = jnp.exp(-jnp.maximum(x, x * ALPHA))
                plsc.store_scatter(srow, [rowv, wcol], w)
                for q in range(16):
                    col = jnp.full((16,), hh * 16 + q, jnp.int32)
                    colv = plsc.load_gather(grow, [rowv, col])
                    plsc.store_scatter(srow, [rowv, col], colv * w)
        pltpu.sync_copy(srow, acc_sh.at[sidx], add=True)
        return carry

    lax.fori_loop(0, EPT // B, chunk, 0)
    plsc.subcore_barrier()

    def out_chunk(k, carry):
        ch = s + k * NS
        pltpu.sync_copy(acc_sh.at[pl.ds(ch * RCH, RCH)],
                        acc_out_hbm.at[pl.ds(coff + ch * RCH, RCH)])
        return carry

    lax.fori_loop(0, nch, out_chunk, 0)


@functools.cache
def _sc_edges():
    return pl.kernel(
        _sc_edges_body,
        out_type=jax.ShapeDtypeStruct((NC * N, AW), jnp.float32),
        mesh=plsc.VectorSubcoreMesh(
            core_axis_name="c", subcore_axis_name="s", num_cores=NC,
            num_subcores=NS),
        compiler_params=pltpu.CompilerParams(use_tc_tiling_on_sc=False,
                                             needs_layout_passes=False),
        scratch_types=[
            pltpu.VMEM_SHARED((N, AW), jnp.float32),
            pltpu.VMEM((B, 16), jnp.float32),
            pltpu.VMEM((B,), jnp.int32),
            pltpu.VMEM((B,), jnp.int32),
            pltpu.VMEM((B,), jnp.int32),
            pltpu.VMEM((B,), jnp.int32),
            pltpu.VMEM((B, AW), jnp.float32),
            pltpu.VMEM((B, AW), jnp.float32),
            pltpu.VMEM((RCH, AW), jnp.float32),
        ],
    )


def kernel(X, adj, W_heads, a_heads, W_out, a_out):
    src = adj[0]
    dst = adj[1]

    W_all = W_heads.transpose(1, 0, 2).reshape(NF, NH * HID)
    blockmask = jnp.kron(jnp.eye(NH, dtype=jnp.float32),
                         jnp.ones((HID, 1), jnp.float32))
    A1 = blockmask * a_heads[:, :HID].reshape(-1)[:, None]
    A2 = blockmask * a_heads[:, HID:].reshape(-1)[:, None]

    haug1, s1p1 = _prep1(X, W_all, A1, A2)
    acc1 = _sc_edges()(haug1.reshape(NC * N, AW), s1p1.reshape(NC * N, 16),
                       src, dst)

    haug2, s1p2 = _mid(acc1.reshape(NC, N, AW), W_out,
                       a_out[:NOUT].reshape(NOUT, 1),
                       a_out[NOUT:].reshape(NOUT, 1))
    acc2 = _sc_edges()(haug2.reshape(NC * N, AW), s1p2.reshape(NC * N, 16),
                       src, dst)

    return _fin(acc2.reshape(NC, N, AW))


# speedup vs baseline: 7.8719x; 1.4328x over previous
import functools

import jax
import jax.numpy as jnp
from jax import lax
from jax.experimental import pallas as pl
from jax.experimental.pallas import tpu as pltpu
from jax.experimental.pallas import tpu_sc as plsc

ALPHA = 0.3
NH = 8
HID = 16
NOUT = 100
N = 10000
NF = 128
E = 320000

NC = 2
NS = 16
AW = 80
EPT = E // NS
B = 80
RCH = 80
NRCH = N // RCH


def _elu(x):
    return jnp.where(x > 0, x, jnp.exp(x) - 1.0)


def _prep1_body(x_ref, w_ref, a1_ref, a2_ref, haug_ref, s1p_ref):
    h = jnp.dot(x_ref[...], w_ref[...], preferred_element_type=jnp.float32)
    s1 = jnp.dot(h, a1_ref[...], preferred_element_type=jnp.float32)
    s2 = jnp.dot(h, a2_ref[...], preferred_element_type=jnp.float32)
    z12 = jnp.zeros((h.shape[0], 12), jnp.float32)
    haug_ref[0] = jnp.concatenate([h[:, :64], s2[:, :4], z12], axis=1)
    haug_ref[1] = jnp.concatenate([h[:, 64:], s2[:, 4:], z12], axis=1)
    s1p_ref[0] = jnp.concatenate([s1[:, :4], z12], axis=1)
    s1p_ref[1] = jnp.concatenate([s1[:, 4:], z12], axis=1)


def _prep1(X, W_all, A1, A2):
    bl = 1000
    return pl.pallas_call(
        _prep1_body,
        grid=(N // bl,),
        in_specs=[
            pl.BlockSpec((bl, NF), lambda i: (i, 0)),
            pl.BlockSpec((NF, NF), lambda i: (0, 0)),
            pl.BlockSpec((NF, NH), lambda i: (0, 0)),
            pl.BlockSpec((NF, NH), lambda i: (0, 0)),
        ],
        out_specs=[
            pl.BlockSpec((NC, bl, AW), lambda i: (0, i, 0)),
            pl.BlockSpec((NC, bl, 16), lambda i: (0, i, 0)),
        ],
        out_shape=[
            jax.ShapeDtypeStruct((NC, N, AW), jnp.float32),
            jax.ShapeDtypeStruct((NC, N, 16), jnp.float32),
        ],
    )(X, W_all, A1, A2)


def _mid_body(acc_ref, wout_ref, a1_ref, a2_ref, haug_ref, s1p_ref):
    acc = acc_ref[...]
    parts = []
    for c in range(2):
        for hh in range(4):
            parts.append(acc[c, :, 16 * hh:16 * hh + 16]
                         / acc[c, :, 64 + hh:65 + hh])
    xc = _elu(jnp.concatenate(parts, axis=1))
    h2 = jnp.dot(xc, wout_ref[...], preferred_element_type=jnp.float32)
    s1b = jnp.dot(h2, a1_ref[...], preferred_element_type=jnp.float32)
    s2b = jnp.dot(h2, a2_ref[...], preferred_element_type=jnp.float32)
    nrow = h2.shape[0]
    s1b4 = jnp.broadcast_to(s1b, (nrow, 4))
    s2b4 = jnp.broadcast_to(s2b, (nrow, 4))
    z12 = jnp.zeros((nrow, 12), jnp.float32)
    z28 = jnp.zeros((nrow, 28), jnp.float32)
    haug_ref[0] = jnp.concatenate([h2[:, :64], s2b4, z12], axis=1)
    haug_ref[1] = jnp.concatenate([h2[:, 64:100], z28, s2b4, z12], axis=1)
    s1p_ref[0] = jnp.concatenate([s1b4, z12], axis=1)
    s1p_ref[1] = jnp.concatenate([s1b4, z12], axis=1)


def _mid(acc1, W_out, a1c, a2c):
    bl = 1000
    return pl.pallas_call(
        _mid_body,
        grid=(N // bl,),
        in_specs=[
            pl.BlockSpec((NC, bl, AW), lambda i: (0, i, 0)),
            pl.BlockSpec((NF, NOUT), lambda i: (0, 0)),
            pl.BlockSpec((NOUT, 1), lambda i: (0, 0)),
            pl.BlockSpec((NOUT, 1), lambda i: (0, 0)),
        ],
        out_specs=[
            pl.BlockSpec((NC, bl, AW), lambda i: (0, i, 0)),
            pl.BlockSpec((NC, bl, 16), lambda i: (0, i, 0)),
        ],
        out_shape=[
            jax.ShapeDtypeStruct((NC, N, AW), jnp.float32),
            jax.ShapeDtypeStruct((NC, N, 16), jnp.float32),
        ],
    )(acc1, W_out, a1c, a2c)


def _fin_body(acc_ref, out_ref):
    acc = acc_ref[...]
    o0 = acc[0, :, :64] / acc[0, :, 64:65]
    o1 = acc[1, :, :36] / acc[1, :, 64:65]
    out_ref[...] = _elu(jnp.concatenate([o0, o1], axis=1))


def _fin(acc2):
    bl = 1000
    return pl.pallas_call(
        _fin_body,
        grid=(N // bl,),
        in_specs=[pl.BlockSpec((NC, bl, AW), lambda i: (0, i, 0))],
        out_specs=pl.BlockSpec((bl, NOUT), lambda i: (i, 0)),
        out_shape=jax.ShapeDtypeStruct((N, NOUT), jnp.float32),
    )(acc2)


CPT = EPT // B
NJ = B // 16


def _sc_edges_body(haug_hbm, s1p_hbm, src_hbm, dst_hbm, acc_out_hbm,
                   acc_sh, sidx2d, didx2d,
                   sadj0, sadj1, dadj0, dadj1,
                   grow0, grow1, srow0, srow1, s1b0, s1b1, zbuf,
                   gd0, gd1, gs0, gs1, sc0, sc1):
    c = lax.axis_index("c")
    s = lax.axis_index("s")
    coff = c * N

    pltpu.sync_copy(src_hbm.at[pl.ds(s * CPT, CPT)], sidx2d)
    pltpu.sync_copy(dst_hbm.at[pl.ds(s * CPT, CPT)], didx2d)

    zf = jnp.zeros((16,), jnp.float32)
    for r in range(16):
        for q in range(AW // 16):
            zbuf[r, pl.ds(q * 16, 16)] = zf
    nch = jnp.where(s <= (NRCH % NS) - 1, NRCH // NS + 1, NRCH // NS)

    def zero_chunk(k, carry):
        ch = s + k * NS
        for m in range(RCH // 16):
            pltpu.sync_copy(zbuf, acc_sh.at[pl.ds(ch * RCH + m * 16, 16)])
        return carry

    lax.fori_loop(0, nch, zero_chunk, 0)
    plsc.subcore_barrier()

    iota16 = lax.iota(jnp.int32, 16)

    def adjust(t, sadj, dadj):
        for q in range(NJ):
            sl = pl.ds(q * 16, 16)
            sadj[sl] = sidx2d[t, sl] + coff
            dadj[sl] = didx2d[t, sl] + coff

    def issue_gather(sadj, dadj, grow, s1b, gd, gs):
        pltpu.async_copy(haug_hbm.at[dadj], grow, gd)
        pltpu.async_copy(s1p_hbm.at[sadj], s1b, gs)

    def wait_gather(grow, s1b, gd, gs):
        pltpu.make_async_copy(haug_hbm.at[pl.ds(0, B)], grow, gd).wait()
        pltpu.make_async_copy(s1p_hbm.at[pl.ds(0, B)], s1b, gs).wait()

    def wait_scatter(srow, sc):
        pltpu.make_async_copy(srow, acc_sh.at[sidx2d.at[0]], sc).wait()

    def compute_and_scatter(t, grow, s1b, srow, sc):
        for j in range(NJ):
            rowv = j * 16 + iota16
            for hh in range(4):
                hsel = jnp.full((16,), hh, jnp.int32)
                wcol = jnp.full((16,), 64 + hh, jnp.int32)
                s1v = plsc.load_gather(s1b, [rowv, hsel])
                s2v = plsc.load_gather(grow, [rowv, wcol])
                x = s1v + s2v
                w = jnp.exp(-jnp.maximum(x, x * ALPHA))
                plsc.store_scatter(srow, [rowv, wcol], w)
                for q in range(16):
                    col = jnp.full((16,), hh * 16 + q, jnp.int32)
                    colv = plsc.load_gather(grow, [rowv, col])
                    plsc.store_scatter(srow, [rowv, col], colv * w)
        pltpu.async_copy(srow, acc_sh.at[sidx2d.at[t]], sc, add=True)

    adjust(0, sadj0, dadj0)
    issue_gather(sadj0, dadj0, grow0, s1b0, gd0, gs0)

    def pair(i2, carry):
        a = 2 * i2
        b = a + 1
        adjust(b, sadj1, dadj1)
        issue_gather(sadj1, dadj1, grow1, s1b1, gd1, gs1)
        wait_gather(grow0, s1b0, gd0, gs0)

        @pl.when(i2 > 0)
        def _():
            wait_scatter(srow0, sc0)

        compute_and_scatter(a, grow0, s1b0, srow0, sc0)

        @pl.when(i2 < CPT // 2 - 1)
        def _():
            adjust(a + 2, sadj0, dadj0)
            issue_gather(sadj0, dadj0, grow0, s1b0, gd0, gs0)

        wait_gather(grow1, s1b1, gd1, gs1)

        @pl.when(i2 > 0)
        def _():
            wait_scatter(srow1, sc1)

        compute_and_scatter(b, grow1, s1b1, srow1, sc1)
        return carry

    lax.fori_loop(0, CPT // 2, pair, 0)
    wait_scatter(srow0, sc0)
    wait_scatter(srow1, sc1)
    plsc.subcore_barrier()

    def out_chunk(k, carry):
        ch = s + k * NS
        pltpu.sync_copy(acc_sh.at[pl.ds(ch * RCH, RCH)],
                        acc_out_hbm.at[pl.ds(coff + ch * RCH, RCH)])
        return carry

    lax.fori_loop(0, nch, out_chunk, 0)


@functools.cache
def _sc_edges():
    return pl.kernel(
        _sc_edges_body,
        out_type=jax.ShapeDtypeStruct((NC * N, AW), jnp.float32),
        mesh=plsc.VectorSubcoreMesh(
            core_axis_name="c", subcore_axis_name="s", num_cores=NC,
            num_subcores=NS),
        compiler_params=pltpu.CompilerParams(use_tc_tiling_on_sc=False,
                                             needs_layout_passes=False),
        scratch_types=[
            pltpu.VMEM_SHARED((N, AW), jnp.float32),
            pltpu.VMEM((CPT, B), jnp.int32),
            pltpu.VMEM((CPT, B), jnp.int32),
            pltpu.VMEM((B,), jnp.int32),
            pltpu.VMEM((B,), jnp.int32),
            pltpu.VMEM((B,), jnp.int32),
            pltpu.VMEM((B,), jnp.int32),
            pltpu.VMEM((B, AW), jnp.float32),
            pltpu.VMEM((B, AW), jnp.float32),
            pltpu.VMEM((B, AW), jnp.float32),
            pltpu.VMEM((B, AW), jnp.float32),
            pltpu.VMEM((B, 16), jnp.float32),
            pltpu.VMEM((B, 16), jnp.float32),
            pltpu.VMEM((16, AW), jnp.float32),
            pltpu.SemaphoreType.DMA,
            pltpu.SemaphoreType.DMA,
            pltpu.SemaphoreType.DMA,
            pltpu.SemaphoreType.DMA,
            pltpu.SemaphoreType.DMA,
            pltpu.SemaphoreType.DMA,
        ],
    )


def kernel(X, adj, W_heads, a_heads, W_out, a_out):
    src = adj[0].reshape(E // B, B)
    dst = adj[1].reshape(E // B, B)

    W_all = W_heads.transpose(1, 0, 2).reshape(NF, NH * HID)
    blockmask = jnp.kron(jnp.eye(NH, dtype=jnp.float32),
                         jnp.ones((HID, 1), jnp.float32))
    A1 = blockmask * a_heads[:, :HID].reshape(-1)[:, None]
    A2 = blockmask * a_heads[:, HID:].reshape(-1)[:, None]

    haug1, s1p1 = _prep1(X, W_all, A1, A2)
    acc1 = _sc_edges()(haug1.reshape(NC * N, AW), s1p1.reshape(NC * N, 16),
                       src, dst)

    haug2, s1p2 = _mid(acc1.reshape(NC, N, AW), W_out,
                       a_out[:NOUT].reshape(NOUT, 1),
                       a_out[NOUT:].reshape(NOUT, 1))
    acc2 = _sc_edges()(haug2.reshape(NC * N, AW), s1p2.reshape(NC * N, 16),
                       src, dst)

    return _fin(acc2.reshape(NC, N, AW))


# speedup vs baseline: 18.4130x; 2.3391x over previous
import functools

import jax
import jax.numpy as jnp
from jax import lax
from jax.experimental import pallas as pl
from jax.experimental.pallas import tpu as pltpu
from jax.experimental.pallas import tpu_sc as plsc

ALPHA = 0.3
NH = 8
HID = 16
NOUT = 100
N = 10000
NF = 128
E = 320000

NC = 2
NS = 16
AW = 80
EPT = E // NS
B = 80
RCH = 80
NRCH = N // RCH


def _elu(x):
    return jnp.where(x > 0, x, jnp.exp(x) - 1.0)


def _prep1_body(x_ref, w_ref, a1_ref, a2_ref, haug_ref, s1p_ref):
    h = jnp.dot(x_ref[...], w_ref[...], preferred_element_type=jnp.float32)
    s1 = jnp.dot(h, a1_ref[...], preferred_element_type=jnp.float32)
    s2 = jnp.dot(h, a2_ref[...], preferred_element_type=jnp.float32)
    z12 = jnp.zeros((h.shape[0], 12), jnp.float32)
    haug_ref[0] = jnp.concatenate([h[:, :64], s2[:, :4], z12], axis=1)
    haug_ref[1] = jnp.concatenate([h[:, 64:], s2[:, 4:], z12], axis=1)
    s1p_ref[0] = jnp.concatenate([s1[:, :4], z12], axis=1)
    s1p_ref[1] = jnp.concatenate([s1[:, 4:], z12], axis=1)


def _prep1(X, W_all, A1, A2):
    bl = 1000
    return pl.pallas_call(
        _prep1_body,
        grid=(N // bl,),
        in_specs=[
            pl.BlockSpec((bl, NF), lambda i: (i, 0)),
            pl.BlockSpec((NF, NF), lambda i: (0, 0)),
            pl.BlockSpec((NF, NH), lambda i: (0, 0)),
            pl.BlockSpec((NF, NH), lambda i: (0, 0)),
        ],
        out_specs=[
            pl.BlockSpec((NC, bl, AW), lambda i: (0, i, 0)),
            pl.BlockSpec((NC, bl, 16), lambda i: (0, i, 0)),
        ],
        out_shape=[
            jax.ShapeDtypeStruct((NC, N, AW), jnp.float32),
            jax.ShapeDtypeStruct((NC, N, 16), jnp.float32),
        ],
    )(X, W_all, A1, A2)


def _mid_body(acc_ref, wout_ref, a1_ref, a2_ref, haug_ref, s1p_ref):
    acc = acc_ref[...]
    parts = []
    for c in range(2):
        for hh in range(4):
            parts.append(acc[c, :, 16 * hh:16 * hh + 16]
                         / acc[c, :, 64 + hh:65 + hh])
    xc = _elu(jnp.concatenate(parts, axis=1))
    h2 = jnp.dot(xc, wout_ref[...], preferred_element_type=jnp.float32)
    s1b = jnp.dot(h2, a1_ref[...], preferred_element_type=jnp.float32)
    s2b = jnp.dot(h2, a2_ref[...], preferred_element_type=jnp.float32)
    nrow = h2.shape[0]
    s1b4 = jnp.broadcast_to(s1b, (nrow, 4))
    s2b4 = jnp.broadcast_to(s2b, (nrow, 4))
    z12 = jnp.zeros((nrow, 12), jnp.float32)
    z28 = jnp.zeros((nrow, 28), jnp.float32)
    haug_ref[0] = jnp.concatenate([h2[:, :64], s2b4, z12], axis=1)
    haug_ref[1] = jnp.concatenate([h2[:, 64:100], z28, s2b4, z12], axis=1)
    s1p_ref[0] = jnp.concatenate([s1b4, z12], axis=1)
    s1p_ref[1] = jnp.concatenate([s1b4, z12], axis=1)


def _mid(acc1, W_out, a1c, a2c):
    bl = 1000
    return pl.pallas_call(
        _mid_body,
        grid=(N // bl,),
        in_specs=[
            pl.BlockSpec((NC, bl, AW), lambda i: (0, i, 0)),
            pl.BlockSpec((NF, NOUT), lambda i: (0, 0)),
            pl.BlockSpec((NOUT, 1), lambda i: (0, 0)),
            pl.BlockSpec((NOUT, 1), lambda i: (0, 0)),
        ],
        out_specs=[
            pl.BlockSpec((NC, bl, AW), lambda i: (0, i, 0)),
            pl.BlockSpec((NC, bl, 16), lambda i: (0, i, 0)),
        ],
        out_shape=[
            jax.ShapeDtypeStruct((NC, N, AW), jnp.float32),
            jax.ShapeDtypeStruct((NC, N, 16), jnp.float32),
        ],
    )(acc1, W_out, a1c, a2c)


def _fin_body(acc_ref, out_ref):
    acc = acc_ref[...]
    o0 = acc[0, :, :64] / acc[0, :, 64:65]
    o1 = acc[1, :, :36] / acc[1, :, 64:65]
    out_ref[...] = _elu(jnp.concatenate([o0, o1], axis=1))


def _fin(acc2):
    bl = 1000
    return pl.pallas_call(
        _fin_body,
        grid=(N // bl,),
        in_specs=[pl.BlockSpec((NC, bl, AW), lambda i: (0, i, 0))],
        out_specs=pl.BlockSpec((bl, NOUT), lambda i: (i, 0)),
        out_shape=jax.ShapeDtypeStruct((N, NOUT), jnp.float32),
    )(acc2)


CPT = EPT // B
NJ = B // 16


def _sc_edges_body(haug_hbm, s1p_hbm, src_hbm, dst_hbm, acc_out_hbm,
                   acc_sh, sidx2d, didx2d,
                   sadj0, sadj1, dadj0, dadj1,
                   grow0, grow1, srow0, srow1, s1b0, s1b1, zbuf,
                   gd0, gd1, gs0, gs1, sc0, sc1):
    c = lax.axis_index("c")
    s = lax.axis_index("s")
    coff = c * N

    pltpu.sync_copy(src_hbm.at[pl.ds(s * CPT, CPT)], sidx2d)
    pltpu.sync_copy(dst_hbm.at[pl.ds(s * CPT, CPT)], didx2d)

    zf = jnp.zeros((16,), jnp.float32)
    for r in range(16):
        for q in range(AW // 16):
            zbuf[r, pl.ds(q * 16, 16)] = zf
    nch = jnp.where(s <= (NRCH % NS) - 1, NRCH // NS + 1, NRCH // NS)

    def zero_chunk(k, carry):
        ch = s + k * NS
        for m in range(RCH // 16):
            pltpu.sync_copy(zbuf, acc_sh.at[pl.ds(ch * RCH + m * 16, 16)])
        return carry

    lax.fori_loop(0, nch, zero_chunk, 0)
    plsc.subcore_barrier()

    iota16 = lax.iota(jnp.int32, 16)

    def adjust(t, sadj, dadj):
        for q in range(NJ):
            sl = pl.ds(q * 16, 16)
            sadj[sl] = sidx2d[t, sl] + coff
            dadj[sl] = didx2d[t, sl] + coff

    def issue_gather(sadj, dadj, grow, s1b, gd, gs):
        pltpu.async_copy(haug_hbm.at[dadj], grow, gd)
        pltpu.async_copy(s1p_hbm.at[sadj], s1b, gs)

    def wait_gather(grow, s1b, gd, gs):
        pltpu.make_async_copy(haug_hbm.at[pl.ds(0, B)], grow, gd).wait()
        pltpu.make_async_copy(s1p_hbm.at[pl.ds(0, B)], s1b, gs).wait()

    def wait_scatter(srow, sc):
        pltpu.make_async_copy(srow, acc_sh.at[sidx2d.at[0]], sc).wait()

    def compute_and_scatter(t, grow, s1b, srow, sc):
        for j in range(NJ):
            rowv = j * 16 + iota16
            ws = []
            for hh in range(4):
                hsel = jnp.full((16,), hh, jnp.int32)
                wcol = jnp.full((16,), 64 + hh, jnp.int32)
                s1v = plsc.load_gather(s1b, [rowv, hsel])
                s2v = plsc.load_gather(grow, [rowv, wcol])
                x = s1v + s2v
                ws.append(jnp.exp(-jnp.maximum(x, x * ALPHA)))
            for hh in range(4):
                wcol = jnp.full((16,), 64 + hh, jnp.int32)
                plsc.store_scatter(srow, [rowv, wcol], ws[hh])
            for hh in range(4):
                cols = [plsc.load_gather(
                            grow, [rowv, jnp.full((16,), hh * 16 + q,
                                                  jnp.int32)])
                        for q in range(16)]
                prods = [cv * ws[hh] for cv in cols]
                for q in range(16):
                    plsc.store_scatter(
                        srow, [rowv, jnp.full((16,), hh * 16 + q, jnp.int32)],
                        prods[q])
        pltpu.async_copy(srow, acc_sh.at[sidx2d.at[t]], sc, add=True)

    adjust(0, sadj0, dadj0)
    issue_gather(sadj0, dadj0, grow0, s1b0, gd0, gs0)

    def pair(i2, carry):
        a = 2 * i2
        b = a + 1
        adjust(b, sadj1, dadj1)
        issue_gather(sadj1, dadj1, grow1, s1b1, gd1, gs1)
        wait_gather(grow0, s1b0, gd0, gs0)

        @pl.when(i2 > 0)
        def _():
            wait_scatter(srow0, sc0)

        compute_and_scatter(a, grow0, s1b0, srow0, sc0)

        @pl.when(i2 < CPT // 2 - 1)
        def _():
            adjust(a + 2, sadj0, dadj0)
            issue_gather(sadj0, dadj0, grow0, s1b0, gd0, gs0)

        wait_gather(grow1, s1b1, gd1, gs1)

        @pl.when(i2 > 0)
        def _():
            wait_scatter(srow1, sc1)

        compute_and_scatter(b, grow1, s1b1, srow1, sc1)
        return carry

    lax.fori_loop(0, CPT // 2, pair, 0)
    wait_scatter(srow0, sc0)
    wait_scatter(srow1, sc1)
    plsc.subcore_barrier()

    def out_chunk(k, carry):
        ch = s + k * NS
        pltpu.sync_copy(acc_sh.at[pl.ds(ch * RCH, RCH)],
                        acc_out_hbm.at[pl.ds(coff + ch * RCH, RCH)])
        return carry

    lax.fori_loop(0, nch, out_chunk, 0)


@functools.cache
def _sc_edges():
    return pl.kernel(
        _sc_edges_body,
        out_type=jax.ShapeDtypeStruct((NC * N, AW), jnp.float32),
        mesh=plsc.VectorSubcoreMesh(
            core_axis_name="c", subcore_axis_name="s", num_cores=NC,
            num_subcores=NS),
        compiler_params=pltpu.CompilerParams(use_tc_tiling_on_sc=False,
                                             needs_layout_passes=False),
        scratch_types=[
            pltpu.VMEM_SHARED((N, AW), jnp.float32),
            pltpu.VMEM((CPT, B), jnp.int32),
            pltpu.VMEM((CPT, B), jnp.int32),
            pltpu.VMEM((B,), jnp.int32),
            pltpu.VMEM((B,), jnp.int32),
            pltpu.VMEM((B,), jnp.int32),
            pltpu.VMEM((B,), jnp.int32),
            pltpu.VMEM((B, AW), jnp.float32),
            pltpu.VMEM((B, AW), jnp.float32),
            pltpu.VMEM((B, AW), jnp.float32),
            pltpu.VMEM((B, AW), jnp.float32),
            pltpu.VMEM((B, 16), jnp.float32),
            pltpu.VMEM((B, 16), jnp.float32),
            pltpu.VMEM((16, AW), jnp.float32),
            pltpu.SemaphoreType.DMA,
            pltpu.SemaphoreType.DMA,
            pltpu.SemaphoreType.DMA,
            pltpu.SemaphoreType.DMA,
            pltpu.SemaphoreType.DMA,
            pltpu.SemaphoreType.DMA,
        ],
    )


def kernel(X, adj, W_heads, a_heads, W_out, a_out):
    src = adj[0].reshape(E // B, B)
    dst = adj[1].reshape(E // B, B)

    W_all = W_heads.transpose(1, 0, 2).reshape(NF, NH * HID)
    blockmask = jnp.kron(jnp.eye(NH, dtype=jnp.float32),
                         jnp.ones((HID, 1), jnp.float32))
    A1 = blockmask * a_heads[:, :HID].reshape(-1)[:, None]
    A2 = blockmask * a_heads[:, HID:].reshape(-1)[:, None]

    haug1, s1p1 = _prep1(X, W_all, A1, A2)
    acc1 = _sc_edges()(haug1.reshape(NC * N, AW), s1p1.reshape(NC * N, 16),
                       src, dst)

    haug2, s1p2 = _mid(acc1.reshape(NC, N, AW), W_out,
                       a_out[:NOUT].reshape(NOUT, 1),
                       a_out[NOUT:].reshape(NOUT, 1))
    acc2 = _sc_edges()(haug2.reshape(NC * N, AW), s1p2.reshape(NC * N, 16),
                       src, dst)

    return _fin(acc2.reshape(NC, N, AW))
